# Initial kernel scaffold; baseline (speedup 1.0000x reference)
import jax
import jax.numpy as jnp
from jax.experimental import pallas as pl


def kernel(x, emb, W1, b1, W2, b2):
    raise NotImplementedError("write your pallas kernel here")


# speedup vs baseline: 32.3531x; 32.3531x over previous
import functools

import jax
import jax.numpy as jnp
from jax import lax
from jax.experimental import pallas as pl
from jax.experimental.pallas import tpu as pltpu
from jax.experimental.pallas import tpu_sc as plsc

BATCH = 4096
SEQ = 200
DIM = 32
HIDDEN = 8
PAD_SEQ = 208
CHUNK = PAD_SEQ // 2
LANES = 16

NC = 2
NS = 16
NW = NC * NS
RPW = BATCH // NW


def _sc_pool_body(x_hbm, emb_hbm, out_hbm, idx_v, buf0, buf1, pool_v, sem0, sem1):
    wid = lax.axis_index("s") * NC + lax.axis_index("c")
    base = wid * RPW

    pltpu.sync_copy(x_hbm.at[pl.ds(base, RPW)], idx_v)

    bufs = (buf0, buf1)
    sems = (sem0, sem1)

    for h in range(2):
        pltpu.async_copy(emb_hbm.at[idx_v.at[0, h]], bufs[h], sems[h])

    neg_inf = jnp.full((LANES,), -jnp.inf, dtype=jnp.float32)

    def row_body(r, carry):
        acc_lo = neg_inf
        acc_hi = neg_inf
        nxt = lax.rem(r + 1, RPW)
        for h in range(2):
            buf, sem = bufs[h], sems[h]
            pltpu.make_async_copy(emb_hbm.at[idx_v.at[r, h]], buf, sem).wait()

            def red(t, acc):
                lo = jnp.maximum(acc[0], buf[t, pl.ds(0, LANES)])
                hi = jnp.maximum(acc[1], buf[t, pl.ds(LANES, LANES)])
                return (lo, hi)

            acc_lo, acc_hi = lax.fori_loop(
                0, CHUNK, red, (acc_lo, acc_hi), unroll=8)
            pltpu.async_copy(emb_hbm.at[idx_v.at[nxt, h]], buf, sem)
        pool_v[r, pl.ds(0, LANES)] = acc_lo
        pool_v[r, pl.ds(LANES, LANES)] = acc_hi
        return carry

    lax.fori_loop(0, RPW, row_body, 0)

    for h in range(2):
        pltpu.make_async_copy(emb_hbm.at[idx_v.at[0, h]], bufs[h], sems[h]).wait()

    pltpu.sync_copy(pool_v, out_hbm.at[pl.ds(base, RPW)])


_sc_pool = functools.partial(
    pl.kernel,
    out_type=jax.ShapeDtypeStruct((BATCH, DIM), jnp.float32),
    mesh=plsc.VectorSubcoreMesh(core_axis_name="c", subcore_axis_name="s"),
    scratch_types=[
        pltpu.VMEM((RPW, 2, CHUNK), jnp.int32),
        pltpu.VMEM((CHUNK, DIM), jnp.float32),
        pltpu.VMEM((CHUNK, DIM), jnp.float32),
        pltpu.VMEM((RPW, DIM), jnp.float32),
        pltpu.SemaphoreType.DMA,
        pltpu.SemaphoreType.DMA,
    ],
    compiler_params=pltpu.CompilerParams(use_tc_tiling_on_sc=False),
)(_sc_pool_body)


def _mlp_body(pooled_ref, w1t_ref, b1_ref, w2t_ref, b2_ref, out_ref):
    p = pooled_ref[...]
    h = jnp.dot(p, w1t_ref[...], preferred_element_type=jnp.float32)
    h = jnp.maximum(h + b1_ref[...], 0.0)
    z = jnp.dot(h, w2t_ref[...], preferred_element_type=jnp.float32)
    z = z + b2_ref[...]
    out_ref[...] = 1.0 / (1.0 + jnp.exp(-z))


def kernel(x, emb, W1, b1, W2, b2):
    x = x.astype(jnp.int32)
    x_pad = jnp.concatenate([x, x[:, :PAD_SEQ - SEQ]], axis=1)
    x_pad = x_pad.reshape(BATCH, 2, CHUNK)

    pooled = _sc_pool(x_pad, emb)

    out = pl.pallas_call(
        _mlp_body,
        out_shape=jax.ShapeDtypeStruct((BATCH, 1), jnp.float32),
    )(pooled, W1.T, b1.reshape(1, HIDDEN), W2.T, b2.reshape(1, 1))
    return out
